# baseline (device time: 49404 ns/iter reference)
import jax
import jax.numpy as jnp
from jax import lax
from jax.experimental import pallas as pl
from jax.experimental.pallas import tpu as pltpu

N_DEV = 8
HEADS_PER = 8
SQ = 256
HALF = SQ // 2
SKV = 4096
NBLK = 64
DH = 128
DM = HEADS_PER * DH
BLK = 64
SCALE = 0.08838834764831843

_BASE = {0: 0, 1: 22, 2: 43}
DST_POS = [_BASE[b % 3] + b // 3 for b in range(NBLK)]
R0, R1, R2 = 0, 22 * BLK, 43 * BLK


def kernel(x, Wq, K_ext, V_ext, Wo):
    Kr = K_ext.reshape(NBLK, BLK, 64, DH)
    Vr = V_ext.reshape(NBLK, BLK, 64, DH)

    XOR_STAGES = (1, 3, 4)

    def body(x_ref, wq_ref, k_any, v_any, wo_ref, out_ref,
             kv_bufs, kv_sems, send_ref, recv_ref, send_sems, recv_sems):
        my_pos = lax.axis_index("i")
        partners = [jnp.bitwise_xor(my_pos, c) for c in XOR_STAGES]

        barrier_sem = pltpu.get_barrier_semaphore()
        for nbr in partners:
            pl.semaphore_signal(
                barrier_sem, inc=1,
                device_id=(nbr,), device_id_type=pl.DeviceIdType.MESH,
            )
        pl.semaphore_wait(barrier_sem, len(partners))

        def start_head(h, slot):
            head = my_pos * HEADS_PER + h
            cps = []
            for t, src in ((0, k_any), (1, v_any)):
                for b in range(NBLK):
                    cp = pltpu.make_async_copy(
                        src.at[b, :, head, :],
                        kv_bufs.at[slot, t, DST_POS[b]],
                        kv_sems.at[slot, t])
                    cp.start()
                    cps.append(cp)
            return cps

        pending = start_head(0, 0)

        xb = x_ref[0].astype(jnp.bfloat16)
        wqb = wq_ref[...].astype(jnp.bfloat16)
        wob = wo_ref[...].astype(jnp.bfloat16)
        q = jnp.dot(xb, wqb, preferred_element_type=jnp.float32)
        qs = (q * SCALE).astype(jnp.bfloat16)

        def attend(qrows, segs):
            ws = []
            for kseg, _ in segs:
                s = lax.dot_general(
                    qrows, kseg, (((1,), (1,)), ((), ())),
                    preferred_element_type=jnp.float32)
                ws.append(jnp.exp(s))
            denom = sum(jnp.sum(w, axis=-1, keepdims=True) for w in ws)
            ctx = sum(
                jnp.dot(w.astype(jnp.bfloat16), vseg,
                        preferred_element_type=jnp.float32)
                for w, (_, vseg) in zip(ws, segs))
            return (ctx / denom).astype(jnp.bfloat16)

        acc = jnp.zeros((SQ, DM), jnp.float32)
        for h in range(HEADS_PER):
            slot = h % 2
            for cp in pending:
                cp.wait()
            if h + 1 < HEADS_PER:
                pending = start_head(h + 1, (h + 1) % 2)
            kh = kv_bufs[slot, 0].reshape(SKV, DH).astype(jnp.bfloat16)
            vh = kv_bufs[slot, 1].reshape(SKV, DH).astype(jnp.bfloat16)
            seg = lambda lo, n: (kh[lo:lo + n], vh[lo:lo + n])
            qh = qs[:, h * DH:(h + 1) * DH]
            q03 = jnp.concatenate([qh[0:BLK], qh[3 * BLK:4 * BLK]], 0)
            ctx03 = attend(q03, [seg(R0, 22 * BLK)])
            ctx1 = attend(qh[BLK:2 * BLK],
                          [seg(R0, BLK), seg(R1, BLK), seg(R2, 21 * BLK)])
            ctx2 = attend(qh[2 * BLK:3 * BLK],
                          [seg(R0, BLK), seg(R2, BLK), seg(R1, 21 * BLK)])
            ctx = jnp.concatenate(
                [ctx03[0:BLK], ctx1, ctx2, ctx03[BLK:2 * BLK]], 0)
            acc = acc + jnp.dot(ctx, wob[h * DH:(h + 1) * DH, :],
                                preferred_element_type=jnp.float32)

        def exchange(s, c, a):
            send_ref[s, c] = a.astype(jnp.bfloat16)
            rdma = pltpu.make_async_remote_copy(
                src_ref=send_ref.at[s, c],
                dst_ref=recv_ref.at[s, c],
                send_sem=send_sems.at[s, c],
                recv_sem=recv_sems.at[s, c],
                device_id=(partners[s],),
                device_id_type=pl.DeviceIdType.MESH,
            )
            rdma.start()
            return rdma

        def finish(s, c, a, rdma):
            rdma.wait()
            return a + recv_ref[s, c][...].astype(jnp.float32)

        acc_t = acc[0:HALF]
        acc_b = acc[HALF:SQ]
        x_t = exchange(0, 0, acc_t)
        x_b = exchange(0, 1, acc_b)
        acc_t = finish(0, 0, acc_t, x_t)
        y_t = exchange(1, 0, acc_t)
        acc_b = finish(0, 1, acc_b, x_b)
        y_b = exchange(1, 1, acc_b)
        acc_t = finish(1, 0, acc_t, y_t)
        z_t = exchange(2, 0, acc_t)
        acc_b = finish(1, 1, acc_b, y_b)
        z_b = exchange(2, 1, acc_b)
        acc_t = finish(2, 0, acc_t, z_t)
        out_ref[0, 0:HALF, :] = acc_t
        acc_b = finish(2, 1, acc_b, z_b)
        out_ref[0, HALF:SQ, :] = acc_b

    out = pl.pallas_call(
        body,
        out_shape=jax.ShapeDtypeStruct((1, SQ, DM), jnp.float32),
        in_specs=[
            pl.BlockSpec(memory_space=pltpu.VMEM),
            pl.BlockSpec(memory_space=pltpu.VMEM),
            pl.BlockSpec(memory_space=pl.ANY),
            pl.BlockSpec(memory_space=pl.ANY),
            pl.BlockSpec(memory_space=pltpu.VMEM),
        ],
        out_specs=pl.BlockSpec(memory_space=pltpu.VMEM),
        scratch_shapes=[
            pltpu.VMEM((2, 2, NBLK, BLK, DH), jnp.float32),
            pltpu.SemaphoreType.DMA((2, 2)),
            pltpu.VMEM((3, 2, HALF, DM), jnp.bfloat16),
            pltpu.VMEM((3, 2, HALF, DM), jnp.bfloat16),
            pltpu.SemaphoreType.DMA((3, 2)),
            pltpu.SemaphoreType.DMA((3, 2)),
        ],
        compiler_params=pltpu.CompilerParams(
            collective_id=0, vmem_limit_bytes=62 * 1024 * 1024),
    )(x, Wq, Kr, Vr, Wo)
    return out


# device time: 47700 ns/iter; 1.0357x vs baseline; 1.0357x over previous
import jax
import jax.numpy as jnp
from jax import lax
from jax.experimental import pallas as pl
from jax.experimental.pallas import tpu as pltpu

N_DEV = 8
HEADS_PER = 8
SQ = 256
HALF = SQ // 2
SKV = 4096
NBLK = 64
DH = 128
DM = HEADS_PER * DH
BLK = 64
SCALE = 0.08838834764831843

_BASE = {0: 0, 1: 22, 2: 43}
DST_POS = [_BASE[b % 3] + b // 3 for b in range(NBLK)]
R0, R1, R2 = 0, 22 * BLK, 43 * BLK


def kernel(x, Wq, K_ext, V_ext, Wo):
    Kr = K_ext.reshape(NBLK, BLK, 64, DH)
    Vr = V_ext.reshape(NBLK, BLK, 64, DH)

    XOR_STAGES = (1, 3, 4)

    def body(x_ref, wq_ref, k_any, v_any, wo_ref, out_ref,
             kv_bufs, kv_sems, send_ref, recv_ref, send_sems, recv_sems):
        my_pos = lax.axis_index("i")
        partners = [jnp.bitwise_xor(my_pos, c) for c in XOR_STAGES]

        barrier_sem = pltpu.get_barrier_semaphore()
        for nbr in partners:
            pl.semaphore_signal(
                barrier_sem, inc=1,
                device_id=(nbr,), device_id_type=pl.DeviceIdType.MESH,
            )
        pl.semaphore_wait(barrier_sem, len(partners))

        def start_head(h, slot):
            head = my_pos * HEADS_PER + h
            kcp = pltpu.make_async_copy(
                k_any.at[:, :, head, :], kv_bufs.at[slot, 0],
                kv_sems.at[slot, 0])
            vcp = pltpu.make_async_copy(
                v_any.at[:, :, head, :], kv_bufs.at[slot, 1],
                kv_sems.at[slot, 1])
            kcp.start()
            vcp.start()
            return kcp, vcp

        pending = start_head(0, 0)

        xb = x_ref[0].astype(jnp.bfloat16)
        wqb = wq_ref[...].astype(jnp.bfloat16)
        wob = wo_ref[...].astype(jnp.bfloat16)
        q = jnp.dot(xb, wqb, preferred_element_type=jnp.float32)
        qs = (q * SCALE).astype(jnp.bfloat16)

        def attend(qrows, segs):
            ws = []
            for kseg, _ in segs:
                s = lax.dot_general(
                    qrows, kseg, (((1,), (1,)), ((), ())),
                    preferred_element_type=jnp.float32)
                ws.append(jnp.exp(s))
            denom = sum(jnp.sum(w, axis=-1, keepdims=True) for w in ws)
            ctx = sum(
                jnp.dot(w.astype(jnp.bfloat16), vseg,
                        preferred_element_type=jnp.float32)
                for w, (_, vseg) in zip(ws, segs))
            return (ctx / denom).astype(jnp.bfloat16)

        acc = jnp.zeros((SQ, DM), jnp.float32)
        for h in range(HEADS_PER):
            slot = h % 2
            pending[0].wait()
            pending[1].wait()
            if h + 1 < HEADS_PER:
                pending = start_head(h + 1, (h + 1) % 2)
            def classes(t):
                dense = kv_bufs[slot, t]
                b0 = dense[0].astype(jnp.bfloat16)
                g = dense[1:NBLK].reshape(21, 3, BLK, DH)
                c1 = g[:, 0].reshape(21 * BLK, DH).astype(jnp.bfloat16)
                c2 = g[:, 1].reshape(21 * BLK, DH).astype(jnp.bfloat16)
                c0 = g[:, 2].reshape(21 * BLK, DH).astype(jnp.bfloat16)
                return b0, c0, c1, c2

            kb0, kc0, kc1, kc2 = classes(0)
            vb0, vc0, vc1, vc2 = classes(1)
            qh = qs[:, h * DH:(h + 1) * DH]
            q03 = jnp.concatenate([qh[0:BLK], qh[3 * BLK:4 * BLK]], 0)
            ctx03 = attend(q03, [(kb0, vb0), (kc0, vc0)])
            ctx1 = attend(qh[BLK:2 * BLK],
                          [(kb0, vb0), (kc1[0:BLK], vc1[0:BLK]),
                           (kc2, vc2)])
            ctx2 = attend(qh[2 * BLK:3 * BLK],
                          [(kb0, vb0), (kc2[0:BLK], vc2[0:BLK]),
                           (kc1, vc1)])
            ctx = jnp.concatenate(
                [ctx03[0:BLK], ctx1, ctx2, ctx03[BLK:2 * BLK]], 0)
            acc = acc + jnp.dot(ctx, wob[h * DH:(h + 1) * DH, :],
                                preferred_element_type=jnp.float32)

        def exchange(s, c, a):
            send_ref[s, c] = a.astype(jnp.bfloat16)
            rdma = pltpu.make_async_remote_copy(
                src_ref=send_ref.at[s, c],
                dst_ref=recv_ref.at[s, c],
                send_sem=send_sems.at[s, c],
                recv_sem=recv_sems.at[s, c],
                device_id=(partners[s],),
                device_id_type=pl.DeviceIdType.MESH,
            )
            rdma.start()
            return rdma

        def finish(s, c, a, rdma):
            rdma.wait()
            return a + recv_ref[s, c][...].astype(jnp.float32)

        acc_t = acc[0:HALF]
        acc_b = acc[HALF:SQ]
        x_t = exchange(0, 0, acc_t)
        x_b = exchange(0, 1, acc_b)
        acc_t = finish(0, 0, acc_t, x_t)
        y_t = exchange(1, 0, acc_t)
        acc_b = finish(0, 1, acc_b, x_b)
        y_b = exchange(1, 1, acc_b)
        acc_t = finish(1, 0, acc_t, y_t)
        z_t = exchange(2, 0, acc_t)
        acc_b = finish(1, 1, acc_b, y_b)
        z_b = exchange(2, 1, acc_b)
        acc_t = finish(2, 0, acc_t, z_t)
        out_ref[0, 0:HALF, :] = acc_t
        acc_b = finish(2, 1, acc_b, z_b)
        out_ref[0, HALF:SQ, :] = acc_b

    out = pl.pallas_call(
        body,
        out_shape=jax.ShapeDtypeStruct((1, SQ, DM), jnp.float32),
        in_specs=[
            pl.BlockSpec(memory_space=pltpu.VMEM),
            pl.BlockSpec(memory_space=pltpu.VMEM),
            pl.BlockSpec(memory_space=pl.ANY),
            pl.BlockSpec(memory_space=pl.ANY),
            pl.BlockSpec(memory_space=pltpu.VMEM),
        ],
        out_specs=pl.BlockSpec(memory_space=pltpu.VMEM),
        scratch_shapes=[
            pltpu.VMEM((2, 2, NBLK, BLK, DH), jnp.float32),
            pltpu.SemaphoreType.DMA((2, 2)),
            pltpu.VMEM((3, 2, HALF, DM), jnp.bfloat16),
            pltpu.VMEM((3, 2, HALF, DM), jnp.bfloat16),
            pltpu.SemaphoreType.DMA((3, 2)),
            pltpu.SemaphoreType.DMA((3, 2)),
        ],
        compiler_params=pltpu.CompilerParams(
            collective_id=0, vmem_limit_bytes=62 * 1024 * 1024),
    )(x, Wq, Kr, Vr, Wo)
    return out
